# baseline (speedup 1.0000x reference)
import functools

import jax
import jax.numpy as jnp
from jax import lax
from jax.experimental import pallas as pl
from jax.experimental.pallas import tpu as pltpu
from jax.experimental.pallas import tpu_sc as plsc

VOCAB = 1000000
EMBED_DIM = 64
BATCH = 16384
HIST = 50

_NC, _NS = 2, 16
_NW = _NC * _NS
_IDXW = 128
_SB = BATCH // _NW
_SUB = 256
_NSUB = 2
_L = 16


def _gather_kernel(idx_hbm, table_hbm, out_hbm, idx_v, rows0, rows1,
                   tr0, tr1, gsem, osem):
    wid = lax.axis_index("s") * _NC + lax.axis_index("c")
    col0 = wid * _SB
    irow0 = col0 // _IDXW

    pltpu.sync_copy(idx_hbm.at[:, pl.ds(irow0, _SB // _IDXW)], idx_v)

    iota = lax.iota(jnp.int32, _L)

    def fire_gathers(h, s, rows_v):
        for j in range(_SUB // _IDXW):
            pltpu.async_copy(
                table_hbm.at[idx_v.at[h, 2 * s + j]],
                rows_v.at[pl.ds(j * _IDXW, _IDXW)],
                gsem,
            )

    def drain_gathers(rows_v):
        pltpu.make_async_copy(
            table_hbm.at[pl.ds(0, _SUB)], rows_v, gsem
        ).wait()

    def transpose(rows_v, tr_v):
        def dbody(d, carry):
            dvec = jnp.full((_L,), 0, jnp.int32) + d
            for j0 in range(0, _SUB, _L):
                v = plsc.load_gather(rows_v, [iota + j0, dvec])
                tr_v[d, pl.ds(j0, _L)] = v
            return carry

        lax.fori_loop(0, EMBED_DIM, dbody, 0)

    def out_slice(h, s):
        return out_hbm.at[
            pl.ds(h * EMBED_DIM, EMBED_DIM),
            pl.ds(pl.multiple_of(col0 + s * _SUB, 8), _SUB),
        ]

    def wait_out(h, s, tr_v):
        pltpu.make_async_copy(tr_v, out_slice(h, s), osem).wait()

    def sub_step(h, s, rows_cur, tr_cur, nh, ns, rows_nxt,
                 ph, ps, tr_prv, first, last):
        if not last:
            fire_gathers(nh, ns, rows_nxt)
        drain_gathers(rows_cur)
        transpose(rows_cur, tr_cur)
        if not first:
            wait_out(ph, ps, tr_prv)
        pltpu.async_copy(tr_cur, out_slice(h, s), osem)

    fire_gathers(0, 0, rows0)

    def hbody(h, carry):
        sub_step(h, 0, rows0, tr0, h, 1, rows1, h - 1, 1, tr1,
                 first=False, last=False)
        sub_step(h, 1, rows1, tr1, h + 1, 0, rows0, h, 0, tr0,
                 first=False, last=False)
        return carry

    sub_step(0, 0, rows0, tr0, 0, 1, rows1, 0, 0, tr0, first=True, last=False)
    sub_step(0, 1, rows1, tr1, 1, 0, rows0, 0, 0, tr0, first=False, last=False)
    lax.fori_loop(1, HIST - 1, hbody, 0)
    sub_step(HIST - 1, 0, rows0, tr0, HIST - 1, 1, rows1, HIST - 2, 1, tr1,
             first=False, last=False)
    sub_step(HIST - 1, 1, rows1, tr1, 0, 0, rows0, HIST - 1, 0, tr0,
             first=False, last=True)
    wait_out(HIST - 1, 1, tr1)


def _embed_lookup(idx3, table):
    mesh = plsc.VectorSubcoreMesh(core_axis_name="c", subcore_axis_name="s")
    k = functools.partial(
        pl.kernel,
        mesh=mesh,
        out_type=jax.ShapeDtypeStruct((HIST * EMBED_DIM, BATCH), jnp.float32),
        scratch_types=[
            pltpu.VMEM((HIST, _SB // _IDXW, _IDXW), jnp.int32),
            pltpu.VMEM((_SUB, EMBED_DIM), jnp.float32),
            pltpu.VMEM((_SUB, EMBED_DIM), jnp.float32),
            pltpu.VMEM((EMBED_DIM, _SUB), jnp.float32),
            pltpu.VMEM((EMBED_DIM, _SUB), jnp.float32),
            pltpu.SemaphoreType.DMA,
            pltpu.SemaphoreType.DMA,
        ],
        compiler_params=pltpu.CompilerParams(
            use_tc_tiling_on_sc=False, needs_layout_passes=False
        ),
    )(_gather_kernel)
    return k(idx3, table)


def kernel(x, table):
    idx3 = x.T.reshape(HIST, BATCH // _IDXW, _IDXW).astype(jnp.int32)
    out_phys = _embed_lookup(idx3, table)
    return out_phys.reshape(HIST, EMBED_DIM, BATCH).transpose(2, 0, 1)


# speedup vs baseline: 1.6255x; 1.6255x over previous
import functools

import jax
import jax.numpy as jnp
from jax import lax
from jax.experimental import pallas as pl
from jax.experimental.pallas import tpu as pltpu
from jax.experimental.pallas import tpu_sc as plsc

VOCAB = 1000000
EMBED_DIM = 64
BATCH = 16384
HIST = 50

_B = BATCH * HIST
_NC, _NS = 2, 16
_NW = _NC * _NS
_ROWS_PER_W = _B // _NW
_IDXW = 128
_K = 5
_CHUNK = _K * _IDXW
_N_CHUNKS = _ROWS_PER_W // _CHUNK
_N_PAIRS = _N_CHUNKS // 2


def _gather_kernel(idx_hbm, table_hbm, out_hbm, idx_v, rows0, rows1, gsem):
    wid = lax.axis_index("s") * _NC + lax.axis_index("c")
    row0 = wid * _ROWS_PER_W
    irow0 = row0 // _IDXW

    pltpu.sync_copy(
        idx_hbm.at[pl.ds(pl.multiple_of(irow0, 8), _ROWS_PER_W // _IDXW)], idx_v
    )

    def out_slice(chunk):
        off = pl.multiple_of(row0 + chunk * _CHUNK, 8)
        return out_hbm.at[pl.ds(off, _CHUNK)]

    def fire_gathers(chunk, rows_v):
        for j in range(_K):
            pltpu.async_copy(
                table_hbm.at[idx_v.at[chunk * _K + j]],
                rows_v.at[pl.ds(j * _IDXW, _IDXW)],
                gsem,
            )

    def drain_chunk(rows_v):
        pltpu.make_async_copy(
            table_hbm.at[pl.ds(0, _CHUNK)], rows_v, gsem
        ).wait()

    fire_gathers(0, rows0)

    def pair(i2, fire_ahead):
        c0 = 2 * i2
        c1 = c0 + 1
        fire_gathers(c1, rows1)
        drain_chunk(rows0)
        pltpu.sync_copy(rows0, out_slice(c0))
        if fire_ahead:
            fire_gathers(c0 + 2, rows0)
        drain_chunk(rows1)
        pltpu.sync_copy(rows1, out_slice(c1))

    def body(i2, carry):
        pair(i2, fire_ahead=True)
        return carry

    lax.fori_loop(0, _N_PAIRS - 1, body, 0)
    pair(_N_PAIRS - 1, fire_ahead=False)


def _embed_lookup(idx2d, table):
    mesh = plsc.VectorSubcoreMesh(core_axis_name="c", subcore_axis_name="s")
    k = functools.partial(
        pl.kernel,
        mesh=mesh,
        out_type=jax.ShapeDtypeStruct((_B, EMBED_DIM), jnp.float32),
        scratch_types=[
            pltpu.VMEM((_ROWS_PER_W // _IDXW, _IDXW), jnp.int32),
            pltpu.VMEM((_CHUNK, EMBED_DIM), jnp.float32),
            pltpu.VMEM((_CHUNK, EMBED_DIM), jnp.float32),
            pltpu.SemaphoreType.DMA,
        ],
        compiler_params=pltpu.CompilerParams(use_tc_tiling_on_sc=False),
    )(_gather_kernel)
    return k(idx2d, table)


def kernel(x, table):
    idx2d = x.reshape(_B // _IDXW, _IDXW).astype(jnp.int32)
    out = _embed_lookup(idx2d, table)
    return out.reshape(BATCH, HIST, EMBED_DIM)


# speedup vs baseline: 1.6346x; 1.0056x over previous
import functools

import jax
import jax.numpy as jnp
from jax import lax
from jax.experimental import pallas as pl
from jax.experimental.pallas import tpu as pltpu
from jax.experimental.pallas import tpu_sc as plsc

VOCAB = 1000000
EMBED_DIM = 64
BATCH = 16384
HIST = 50

_B = BATCH * HIST
_NC, _NS = 2, 16
_NW = _NC * _NS
_ROWS_PER_W = _B // _NW
_IDXW = 256
_K = 2
_CHUNK = _K * _IDXW
_N_CHUNKS = _ROWS_PER_W // _CHUNK
_N_PAIRS = _N_CHUNKS // 2


def _gather_kernel(idx_hbm, table_hbm, out_hbm, idx_v, rows0, rows1, gsem):
    wid = lax.axis_index("s") * _NC + lax.axis_index("c")
    row0 = wid * _ROWS_PER_W
    irow0 = row0 // _IDXW

    pltpu.sync_copy(
        idx_hbm.at[pl.ds(pl.multiple_of(irow0, 8), _ROWS_PER_W // _IDXW)], idx_v
    )

    def out_slice(chunk):
        off = pl.multiple_of(row0 + chunk * _CHUNK, 8)
        return out_hbm.at[pl.ds(off, _CHUNK)]

    def fire_gathers(chunk, rows_v):
        for j in range(_K):
            pltpu.async_copy(
                table_hbm.at[idx_v.at[chunk * _K + j]],
                rows_v.at[pl.ds(j * _IDXW, _IDXW)],
                gsem,
            )

    def drain_chunk(rows_v):
        pltpu.make_async_copy(
            table_hbm.at[pl.ds(0, _CHUNK)], rows_v, gsem
        ).wait()

    fire_gathers(0, rows0)

    def pair(i2, fire_ahead):
        c0 = 2 * i2
        c1 = c0 + 1
        fire_gathers(c1, rows1)
        drain_chunk(rows0)
        pltpu.sync_copy(rows0, out_slice(c0))
        if fire_ahead:
            fire_gathers(c0 + 2, rows0)
        drain_chunk(rows1)
        pltpu.sync_copy(rows1, out_slice(c1))

    def body(i2, carry):
        pair(i2, fire_ahead=True)
        return carry

    lax.fori_loop(0, _N_PAIRS - 1, body, 0)
    pair(_N_PAIRS - 1, fire_ahead=False)


def _embed_lookup(idx2d, table):
    mesh = plsc.VectorSubcoreMesh(core_axis_name="c", subcore_axis_name="s")
    k = functools.partial(
        pl.kernel,
        mesh=mesh,
        out_type=jax.ShapeDtypeStruct((_B, EMBED_DIM), jnp.float32),
        scratch_types=[
            pltpu.VMEM((_ROWS_PER_W // _IDXW, _IDXW), jnp.int32),
            pltpu.VMEM((_CHUNK, EMBED_DIM), jnp.float32),
            pltpu.VMEM((_CHUNK, EMBED_DIM), jnp.float32),
            pltpu.SemaphoreType.DMA,
        ],
        compiler_params=pltpu.CompilerParams(use_tc_tiling_on_sc=False),
    )(_gather_kernel)
    return k(idx2d, table)


def kernel(x, table):
    idx2d = x.reshape(_B // _IDXW, _IDXW).astype(jnp.int32)
    out = _embed_lookup(idx2d, table)
    return out.reshape(BATCH, HIST, EMBED_DIM)
